# baseline (speedup 1.0000x reference)
import functools

import jax
import jax.numpy as jnp
from jax import lax
from jax.experimental import pallas as pl
from jax.experimental.pallas import tpu as pltpu
from jax.experimental.pallas import tpu_sc as plsc

_INFO = plsc.get_sparse_core_info()
_NC, _NS = _INFO.num_cores, _INFO.num_subcores
_NW = _NC * _NS

_N, _M, _D = 16384, 200, 64
_IT_W = (_N // 128) // _NW
_IW = _IT_W * 128


_TSTRIDE = 513


def _embed_kernel(xt_hbm, trep_hbm, out5_hbm, table_v, idx_v, tiles_v,
                  isem0, isem1, osem0, osem1):
    wid = lax.axis_index("s") * _NC + lax.axis_index("c")
    it0 = wid * _IT_W
    ibase = wid * _IW
    pltpu.sync_copy(trep_hbm, table_v)
    laneoff = lax.iota(jnp.int32, 16) * _TSTRIDE
    pltpu.async_copy(xt_hbm.at[0, pl.ds(ibase, _IW)], idx_v.at[0], isem0)

    def j_step(j2, p, isem_here, isem_next, osem):
        j = j2 * 2 + p
        pltpu.make_async_copy(
            xt_hbm.at[j, pl.ds(ibase, _IW)], idx_v.at[p], isem_here).wait()

        @pl.when(j + 1 < _M)
        def _():
            pltpu.async_copy(
                xt_hbm.at[j + 1, pl.ds(ibase, _IW)], idx_v.at[1 - p],
                isem_next)

        for kt in range(8):
            @pl.when(jnp.logical_and(j2 >= 1, kt == 0))
            def _(kt=kt):
                pltpu.make_async_copy(
                    tiles_v.at[p, kt],
                    out5_hbm.at[j, kt, pl.ds(it0, _IT_W)], osem).wait()

            def cg_body(cg, carry, kt=kt):
                vals = []
                for itl in range(_IT_W):
                    svec = idx_v[p, pl.ds(itl * 128 + cg * 16, 16)]
                    base = svec * _D + laneoff + (kt * 8)
                    vals.append(
                        [plsc.load_gather(table_v, [base + r])
                         for r in range(8)])
                for itl in range(_IT_W):
                    for r in range(8):
                        tiles_v[p, kt, itl, r, pl.ds(cg * 16, 16)] = (
                            vals[itl][r])
                return carry
            lax.fori_loop(0, 8, cg_body, 0)
            if kt == 0:
                pltpu.async_copy(
                    tiles_v.at[p, kt],
                    out5_hbm.at[j, kt, pl.ds(it0, _IT_W)], osem)

    def outer(j2, carry):
        j_step(j2, 0, isem0, isem1, osem0)
        j_step(j2, 1, isem1, isem0, osem1)
        return carry

    lax.fori_loop(0, _M // 2, outer, 0)

    for p, osem in ((0, osem0), (1, osem1)):
        for kt in range(1):
            pltpu.make_async_copy(
                tiles_v.at[p, kt],
                out5_hbm.at[_M - 2 + p, kt, pl.ds(it0, _IT_W)], osem).wait()


@jax.jit
def kernel(x, table):
    xt = jnp.transpose(x)
    trep = jnp.tile(jnp.append(table.reshape(8 * _D), 0.0), 16)
    mesh = plsc.VectorSubcoreMesh(core_axis_name="c", subcore_axis_name="s")
    run = functools.partial(
        pl.kernel,
        mesh=mesh,
        out_type=jax.ShapeDtypeStruct((_M, 8, _N // 128, 8, 128),
                                      jnp.float32),
        scratch_types=[
            pltpu.VMEM((16 * _TSTRIDE,), jnp.float32),
            pltpu.VMEM((2, _IW), jnp.int32),
            pltpu.VMEM((2, 8, _IT_W, 8, 128), jnp.float32),
            pltpu.SemaphoreType.DMA,
            pltpu.SemaphoreType.DMA,
            pltpu.SemaphoreType.DMA,
            pltpu.SemaphoreType.DMA,
        ],
        compiler_params=pltpu.CompilerParams(
            use_tc_tiling_on_sc=False, needs_layout_passes=False),
    )(_embed_kernel)
    out5 = run(xt, trep)
    return out5.transpose(2, 4, 0, 1, 3).reshape(_N, _M, _D)


# speedup vs baseline: 1.1359x; 1.1359x over previous
import functools

import jax
import jax.numpy as jnp
from jax import lax
from jax.experimental import pallas as pl
from jax.experimental.pallas import tpu as pltpu
from jax.experimental.pallas import tpu_sc as plsc

_INFO = plsc.get_sparse_core_info()
_NC, _NS = _INFO.num_cores, _INFO.num_subcores
_NW = _NC * _NS

_N, _M, _D = 16384, 200, 64
_IT_W = (_N // 128) // _NW
_IW = _IT_W * 128


_TSTRIDE = 513


def _embed_kernel(xt_hbm, trep_hbm, out5_hbm, table_v, idx_v, tiles_v,
                  isem0, isem1, osem0, osem1):
    wid = lax.axis_index("s") * _NC + lax.axis_index("c")
    it0 = wid * _IT_W
    ibase = wid * _IW
    pltpu.sync_copy(trep_hbm, table_v)
    laneoff = lax.iota(jnp.int32, 16) * _TSTRIDE
    pltpu.async_copy(xt_hbm.at[0, pl.ds(ibase, _IW)], idx_v.at[0], isem0)

    def j_step(j2, p, isem_here, isem_next, osem):
        j = j2 * 2 + p
        pltpu.make_async_copy(
            xt_hbm.at[j, pl.ds(ibase, _IW)], idx_v.at[p], isem_here).wait()

        @pl.when(j + 1 < _M)
        def _():
            pltpu.async_copy(
                xt_hbm.at[j + 1, pl.ds(ibase, _IW)], idx_v.at[1 - p],
                isem_next)

        @pl.when(j2 >= 1)
        def _():
            for kt in range(8):
                pltpu.make_async_copy(
                    tiles_v.at[p, kt],
                    out5_hbm.at[j, kt, pl.ds(it0, _IT_W)], osem).wait()

        def cg_body(cg, carry):
            for itl in range(_IT_W):
                svec = idx_v[p, pl.ds(itl * 128 + cg * 16, 16)]
                base = svec * _D + laneoff
                for kth in range(4):
                    kts = (2 * kth, 2 * kth + 1)
                    vals = [plsc.load_gather(table_v, [base + (kt * 8 + r)])
                            for kt in kts for r in range(8)]
                    i = 0
                    for kt in kts:
                        for r in range(8):
                            tiles_v[p, kt, itl, r, pl.ds(cg * 16, 16)] = (
                                vals[i])
                            i += 1
            return carry
        lax.fori_loop(0, 8, cg_body, 0)

        for kt in range(8):
            pltpu.async_copy(
                tiles_v.at[p, kt],
                out5_hbm.at[j, kt, pl.ds(it0, _IT_W)], osem)

    def outer(j2, carry):
        j_step(j2, 0, isem0, isem1, osem0)
        j_step(j2, 1, isem1, isem0, osem1)
        return carry

    lax.fori_loop(0, _M // 2, outer, 0)

    for p, osem in ((0, osem0), (1, osem1)):
        for kt in range(8):
            pltpu.make_async_copy(
                tiles_v.at[p, kt],
                out5_hbm.at[_M - 2 + p, kt, pl.ds(it0, _IT_W)], osem).wait()


@jax.jit
def kernel(x, table):
    xt = jnp.transpose(x)
    trep = jnp.tile(jnp.append(table.reshape(8 * _D), 0.0), 16)
    mesh = plsc.VectorSubcoreMesh(core_axis_name="c", subcore_axis_name="s")
    run = functools.partial(
        pl.kernel,
        mesh=mesh,
        out_type=jax.ShapeDtypeStruct((_M, 8, _N // 128, 8, 128),
                                      jnp.float32),
        scratch_types=[
            pltpu.VMEM((16 * _TSTRIDE,), jnp.float32),
            pltpu.VMEM((2, _IW), jnp.int32),
            pltpu.VMEM((2, 8, _IT_W, 8, 128), jnp.float32),
            pltpu.SemaphoreType.DMA,
            pltpu.SemaphoreType.DMA,
            pltpu.SemaphoreType.DMA,
            pltpu.SemaphoreType.DMA,
        ],
        compiler_params=pltpu.CompilerParams(
            use_tc_tiling_on_sc=False, needs_layout_passes=False),
    )(_embed_kernel)
    out5 = run(xt, trep)
    return out5.transpose(2, 4, 0, 1, 3).reshape(_N, _M, _D)


# speedup vs baseline: 1.6792x; 1.4783x over previous
import functools

import jax
import jax.numpy as jnp
from jax import lax
from jax.experimental import pallas as pl
from jax.experimental.pallas import tpu as pltpu
from jax.experimental.pallas import tpu_sc as plsc

_INFO = plsc.get_sparse_core_info()
_NC, _NS = _INFO.num_cores, _INFO.num_subcores
_NW = _NC * _NS

_N, _M, _D = 16384, 200, 64
_IT_W = (_N // 128) // _NW
_IW = _IT_W * 128


def _embed_kernel(xt_hbm, ttp_hbm, out5_hbm, table_v, idx_v, tiles_v,
                  isem0, isem1, osem0, osem1):
    wid = lax.axis_index("s") * _NC + lax.axis_index("c")
    it0 = wid * _IT_W
    ibase = wid * _IW
    pltpu.sync_copy(ttp_hbm, table_v)
    pltpu.async_copy(xt_hbm.at[0, pl.ds(ibase, _IW)], idx_v.at[0], isem0)

    def j_step(j2, p, isem_here, isem_next, osem):
        j = j2 * 2 + p
        pltpu.make_async_copy(
            xt_hbm.at[j, pl.ds(ibase, _IW)], idx_v.at[p], isem_here).wait()

        @pl.when(j + 1 < _M)
        def _():
            pltpu.async_copy(
                xt_hbm.at[j + 1, pl.ds(ibase, _IW)], idx_v.at[1 - p],
                isem_next)

        @pl.when(j2 >= 1)
        def _():
            for kt in range(8):
                pltpu.make_async_copy(
                    tiles_v.at[p, kt],
                    out5_hbm.at[j, kt, pl.ds(it0, _IT_W)], osem).wait()

        for kt in range(8):
            tcols = [table_v[kt * 8 + r, :] for r in range(8)]

            def cg_body(cg, carry, kt=kt, tcols=tcols):
                for itl in range(_IT_W):
                    svec = idx_v[p, pl.ds(itl * 128 + cg * 16, 16)]
                    vals = [
                        lax.gather(
                            tcols[r], svec.reshape(16, 1),
                            lax.GatherDimensionNumbers(
                                offset_dims=(), collapsed_slice_dims=(0,),
                                start_index_map=(0,)),
                            (1,),
                            mode=lax.GatherScatterMode.PROMISE_IN_BOUNDS)
                        for r in range(8)]
                    for r in range(8):
                        tiles_v[p, kt, itl, r, pl.ds(cg * 16, 16)] = vals[r]
                return carry
            lax.fori_loop(0, 8, cg_body, 0)
            pltpu.async_copy(
                tiles_v.at[p, kt],
                out5_hbm.at[j, kt, pl.ds(it0, _IT_W)], osem)

    def outer(j2, carry):
        j_step(j2, 0, isem0, isem1, osem0)
        j_step(j2, 1, isem1, isem0, osem1)
        return carry

    lax.fori_loop(0, _M // 2, outer, 0)

    for p, osem in ((0, osem0), (1, osem1)):
        for kt in range(8):
            pltpu.make_async_copy(
                tiles_v.at[p, kt],
                out5_hbm.at[_M - 2 + p, kt, pl.ds(it0, _IT_W)], osem).wait()


@jax.jit
def kernel(x, table):
    xt = jnp.transpose(x)
    ttp = jnp.pad(jnp.transpose(table), ((0, 0), (0, 8)))
    mesh = plsc.VectorSubcoreMesh(core_axis_name="c", subcore_axis_name="s")
    run = functools.partial(
        pl.kernel,
        mesh=mesh,
        out_type=jax.ShapeDtypeStruct((_M, 8, _N // 128, 8, 128),
                                      jnp.float32),
        scratch_types=[
            pltpu.VMEM((_D, 16), jnp.float32),
            pltpu.VMEM((2, _IW), jnp.int32),
            pltpu.VMEM((2, 8, _IT_W, 8, 128), jnp.float32),
            pltpu.SemaphoreType.DMA,
            pltpu.SemaphoreType.DMA,
            pltpu.SemaphoreType.DMA,
            pltpu.SemaphoreType.DMA,
        ],
        compiler_params=pltpu.CompilerParams(
            use_tc_tiling_on_sc=False, needs_layout_passes=False),
    )(_embed_kernel)
    out5 = run(xt, ttp)
    return out5.transpose(2, 4, 0, 1, 3).reshape(_N, _M, _D)
